# Initial kernel scaffold; baseline (speedup 1.0000x reference)
import jax
import jax.numpy as jnp
from jax.experimental import pallas as pl


def kernel(x_A, x_B, edge_ab, edge_ba, Ws_A, bs_A, Ws_B, bs_B, Wq_A, bq_A, Wq_B, bq_B, Wk_A, bk_A, Wk_B, bk_B, Wal_A, bal_A, Wal_B, bal_B, War_A, bar_A, War_B, bar_B, Wc_ab, bc_ab, Wc_ba, bc_ba):
    raise NotImplementedError("write your pallas kernel here")


# speedup vs baseline: 5.8025x; 5.8025x over previous
import functools

import jax
import jax.numpy as jnp
from jax import lax
from jax.experimental import pallas as pl
from jax.experimental.pallas import tpu as pltpu
from jax.experimental.pallas import tpu_sc as plsc

_N = 10000
_E = 320000
_F = 128
_NSUB = 16
_EPW = _E // _NSUB
_C = 200
_NCH = _EPW // _C
_NPAD = 10240
_RPW = _NPAD // _NSUB


def _sc_aggregate(x_A, x_B, src_ab, dst_ab, src_ba, dst_ba):
    mesh = plsc.VectorSubcoreMesh(core_axis_name="c", subcore_axis_name="s")

    @functools.partial(
        pl.kernel,
        out_type=[
            jax.ShapeDtypeStruct((_NPAD, _F), jnp.float32),
            jax.ShapeDtypeStruct((_NPAD, _F), jnp.float32),
            jax.ShapeDtypeStruct((_NPAD, _F), jnp.float32),
            jax.ShapeDtypeStruct((_NPAD, _F), jnp.float32),
        ],
        mesh=mesh,
        scratch_types=[
            pltpu.VMEM((_C,), jnp.int32),
            pltpu.VMEM((_C,), jnp.int32),
            pltpu.VMEM((_C, _F), jnp.float32),
            pltpu.VMEM_SHARED((_NPAD, _F), jnp.float32),
            pltpu.SemaphoreType.DMA,
        ],
    )
    def body(xa_hbm, xb_hbm, sab_hbm, dab_hbm, sba_hbm, dba_hbm,
             aggA_hbm, aggB_hbm, degA_hbm, degB_hbm,
             idx_s, idx_d, rows, s_agg, sem):
        cid = lax.axis_index("c")
        sid = lax.axis_index("s")

        def fill_rows(val16):
            def fill(r, _):
                for j in range(_F // 16):
                    rows[r, pl.ds(j * 16, 16)] = val16
                return 0
            lax.fori_loop(0, _C, fill, 0)

        def zero_acc():
            def z(i, _):
                pltpu.sync_copy(rows, s_agg.at[pl.ds(sid * _RPW + i * _C, _C), :])
                return 0
            lax.fori_loop(0, _RPW // _C, z, 0)
            if _RPW % _C:
                pltpu.sync_copy(
                    rows.at[pl.ds(0, _RPW % _C), :],
                    s_agg.at[pl.ds(sid * _RPW + (_RPW // _C) * _C, _RPW % _C), :])

        def writeout(dst_hbm):
            pltpu.sync_copy(s_agg.at[pl.ds(sid * _RPW, _RPW), :],
                            dst_hbm.at[pl.ds(sid * _RPW, _RPW), :])

        fill_rows(jnp.zeros((16,), jnp.float32))
        zero_acc()
        plsc.subcore_barrier()

        def make_agg_chunk(src_hbm, dst_hbm, x_hbm):
            def chunk(i, _):
                base = sid * _EPW + i * _C
                pltpu.sync_copy(src_hbm.at[pl.ds(base, _C)], idx_s)
                pltpu.sync_copy(dst_hbm.at[pl.ds(base, _C)], idx_d)
                pltpu.async_copy(x_hbm.at[idx_s], rows, sem).wait()
                pltpu.sync_copy(rows, s_agg.at[idx_d], add=True)
                return 0
            return chunk

        @pl.when(cid == 0)
        def _():
            lax.fori_loop(0, _NCH, make_agg_chunk(sab_hbm, dab_hbm, xa_hbm), 0)

        @pl.when(cid == 1)
        def _():
            lax.fori_loop(0, _NCH, make_agg_chunk(sba_hbm, dba_hbm, xb_hbm), 0)

        plsc.subcore_barrier()

        @pl.when(cid == 0)
        def _():
            writeout(aggB_hbm)

        @pl.when(cid == 1)
        def _():
            writeout(aggA_hbm)

        plsc.subcore_barrier()

        fill_rows(jnp.zeros((16,), jnp.float32))
        zero_acc()
        plsc.subcore_barrier()
        fill_rows(jnp.full((16,), 1.0, jnp.float32))

        def make_deg_chunk(dst_hbm):
            def chunk(i, _):
                base = sid * _EPW + i * _C
                pltpu.sync_copy(dst_hbm.at[pl.ds(base, _C)], idx_d)
                pltpu.sync_copy(rows, s_agg.at[idx_d], add=True)
                return 0
            return chunk

        @pl.when(cid == 0)
        def _():
            lax.fori_loop(0, _NCH, make_deg_chunk(dab_hbm), 0)

        @pl.when(cid == 1)
        def _():
            lax.fori_loop(0, _NCH, make_deg_chunk(dba_hbm), 0)

        plsc.subcore_barrier()

        @pl.when(cid == 0)
        def _():
            writeout(degB_hbm)

        @pl.when(cid == 1)
        def _():
            writeout(degA_hbm)

    return body(x_A, x_B, src_ab, dst_ab, src_ba, dst_ba)


def _tc_dense(x_A, x_B, agg_A, agg_B, deg_A, deg_B,
              Ws_A, Ws_B, Wc_ab, Wc_ba, bs_A, bs_B, bc_ab, bc_ba,
              wl_A, wr_A, wl_B, wr_B, consts):
    BLK = 1000
    grid = _N // BLK

    def elu(x):
        return jnp.where(x > 0, x, jnp.exp(jnp.minimum(x, 0.0)) - 1.0)

    def body(xa, xb, ga, gb, da, db, WsA, WsB, Wcab, Wcba,
             bsA, bsB, bcab, bcba, wlA, wrA, wlB, wrB, cst, out):
        clA, crA, clB, crB = cst[0, 0], cst[0, 1], cst[0, 2], cst[0, 3]
        dot = functools.partial(jnp.dot, precision=jax.lax.Precision.HIGHEST,
                                preferred_element_type=jnp.float32)
        za = dot(xa[...], WsA[...]) + bsA[...]
        zb = dot(xb[...], WsB[...]) + bsB[...]
        hlA = jnp.sum(za * wlA[...], axis=1, keepdims=True) + clA
        hrA = jnp.sum(za * wrA[...], axis=1, keepdims=True) + crA
        hlB = jnp.sum(zb * wlB[...], axis=1, keepdims=True) + clB
        hrB = jnp.sum(zb * wrB[...], axis=1, keepdims=True) + crB
        dA = dot(ga[...] / jnp.maximum(da[...], 1.0), Wcba[...]) + bcba[...]
        dB = dot(gb[...] / jnp.maximum(db[...], 1.0), Wcab[...]) + bcab[...]
        attsA = elu(hlA + hrA)
        attsB = elu(hlB + hrB)
        eA = elu(jnp.sum(dA * wlA[...], axis=1, keepdims=True) + clA + hrA)
        eB = elu(jnp.sum(dB * wlB[...], axis=1, keepdims=True) + clB + hrB)
        mA = jnp.maximum(attsA, eA)
        p0A, p1A = jnp.exp(attsA - mA), jnp.exp(eA - mA)
        sA = p0A + p1A
        mB = jnp.maximum(attsB, eB)
        p0B, p1B = jnp.exp(attsB - mB), jnp.exp(eB - mB)
        sB = p0B + p1B
        out[0] = elu(za * (p0A / sA) + dA * (p1A / sA))
        out[1] = elu(zb * (p0B / sB) + dB * (p1B / sB))

    row_blk = pl.BlockSpec((BLK, _F), lambda i: (i, 0))
    col_blk = pl.BlockSpec((BLK, 1), lambda i: (i, 0))
    w_blk = pl.BlockSpec((_F, _F), lambda i: (0, 0))
    v_blk = pl.BlockSpec((1, _F), lambda i: (0, 0))
    c_blk = pl.BlockSpec((1, 4), lambda i: (0, 0))

    return pl.pallas_call(
        body,
        grid=(grid,),
        in_specs=[row_blk, row_blk, row_blk, row_blk, col_blk, col_blk,
                  w_blk, w_blk, w_blk, w_blk,
                  v_blk, v_blk, v_blk, v_blk,
                  v_blk, v_blk, v_blk, v_blk, c_blk],
        out_specs=pl.BlockSpec((2, BLK, _F), lambda i: (0, i, 0)),
        out_shape=jax.ShapeDtypeStruct((2, _N, _F), jnp.float32),
    )(x_A, x_B, agg_A, agg_B, deg_A, deg_B,
      Ws_A, Ws_B, Wc_ab, Wc_ba, bs_A, bs_B, bc_ab, bc_ba,
      wl_A, wr_A, wl_B, wr_B, consts)


def kernel(x_A, x_B, edge_ab, edge_ba, Ws_A, bs_A, Ws_B, bs_B, Wq_A, bq_A,
           Wq_B, bq_B, Wk_A, bk_A, Wk_B, bk_B, Wal_A, bal_A, Wal_B, bal_B,
           War_A, bar_A, War_B, bar_B, Wc_ab, bc_ab, Wc_ba, bc_ba):
    wl_A = (Wk_A @ Wal_A).reshape(1, _F)
    wl_B = (Wk_B @ Wal_B).reshape(1, _F)
    wr_A = (Wq_A @ War_A).reshape(1, _F)
    wr_B = (Wq_B @ War_B).reshape(1, _F)
    cl_A = bk_A @ Wal_A + bal_A
    cl_B = bk_B @ Wal_B + bal_B
    cr_A = bq_A @ War_A + bar_A
    cr_B = bq_B @ War_B + bar_B
    consts = jnp.concatenate([cl_A, cr_A, cl_B, cr_B]).reshape(1, 4)

    agg_A, agg_B, deg_A, deg_B = _sc_aggregate(
        x_A, x_B, edge_ab[0], edge_ab[1], edge_ba[0], edge_ba[1])

    return _tc_dense(
        x_A, x_B, agg_A[:_N], agg_B[:_N],
        deg_A[:_N, :1], deg_B[:_N, :1],
        Ws_A, Ws_B, Wc_ab, Wc_ba,
        bs_A.reshape(1, _F), bs_B.reshape(1, _F),
        bc_ab.reshape(1, _F), bc_ba.reshape(1, _F),
        wl_A, wr_A, wl_B, wr_B, consts)


# speedup vs baseline: 7.9000x; 1.3615x over previous
import functools

import jax
import jax.numpy as jnp
from jax import lax
from jax.experimental import pallas as pl
from jax.experimental.pallas import tpu as pltpu
from jax.experimental.pallas import tpu_sc as plsc

_N = 10000
_E = 320000
_F = 128
_NSUB = 16
_EPW = _E // _NSUB
_C = 160
_NCH = _EPW // _C
_NPAD = 10240
_RPW = _NPAD // _NSUB


_HALF = (_NCH - 1) // 2


def _sc_aggregate(x_A, x_B, src_ab, dst_ab, src_ba, dst_ba):
    mesh = plsc.VectorSubcoreMesh(core_axis_name="c", subcore_axis_name="s")

    @functools.partial(
        pl.kernel,
        out_type=[
            jax.ShapeDtypeStruct((_NPAD, _F), jnp.float32),
            jax.ShapeDtypeStruct((_NPAD, _F), jnp.float32),
            jax.ShapeDtypeStruct((_NPAD, _F), jnp.float32),
            jax.ShapeDtypeStruct((_NPAD, _F), jnp.float32),
        ],
        mesh=mesh,
        scratch_types=[
            pltpu.VMEM((_C,), jnp.int32),
            pltpu.VMEM((_C,), jnp.int32),
            pltpu.VMEM((_C,), jnp.int32),
            pltpu.VMEM((_C,), jnp.int32),
            pltpu.VMEM((_C, _F), jnp.float32),
            pltpu.VMEM((_C, _F), jnp.float32),
            pltpu.VMEM_SHARED((_NPAD, _F), jnp.float32),
            pltpu.SemaphoreType.DMA,
            pltpu.SemaphoreType.DMA,
        ],
    )
    def body(xa_hbm, xb_hbm, sab_hbm, dab_hbm, sba_hbm, dba_hbm,
             aggA_hbm, aggB_hbm, degA_hbm, degB_hbm,
             is0, id0, is1, id1, rows0, rows1, s_agg, sem0, sem1):
        cid = lax.axis_index("c")
        sid = lax.axis_index("s")

        def fill_rows0(val16):
            def fill(r, _):
                for j in range(_F // 16):
                    rows0[r, pl.ds(j * 16, 16)] = val16
                return 0
            lax.fori_loop(0, _C, fill, 0)

        def zero_acc(s_agg):
            def z(i, _):
                pltpu.sync_copy(rows0,
                                s_agg.at[pl.ds(sid * _RPW + i * _C, _C), :])
                return 0
            lax.fori_loop(0, _RPW // _C, z, 0)

        def writeout(s_agg, dst_hbm):
            pltpu.sync_copy(s_agg.at[pl.ds(sid * _RPW, _RPW), :],
                            dst_hbm.at[pl.ds(sid * _RPW, _RPW), :])

        def run(src_hbm, dst_hbm, x_hbm, agg_hbm, deg_hbm):
            def load_pair(k, is_b, id_b):
                base = sid * _EPW + k * _C
                pltpu.sync_copy(src_hbm.at[pl.ds(base, _C)], is_b)
                pltpu.sync_copy(dst_hbm.at[pl.ds(base, _C)], id_b)

            def load_dst(k, id_b):
                base = sid * _EPW + k * _C
                pltpu.sync_copy(dst_hbm.at[pl.ds(base, _C)], id_b)

            fill_rows0(jnp.zeros((16,), jnp.float32))
            zero_acc(s_agg)
            plsc.subcore_barrier()

            load_pair(0, is0, id0)
            pltpu.async_copy(x_hbm.at[is0], rows0, sem0)
            load_pair(1, is1, id1)
            pltpu.async_copy(x_hbm.at[is1], rows1, sem1)

            def p1(j, _):
                pltpu.make_async_copy(x_hbm.at[is0], rows0, sem0).wait()
                pltpu.sync_copy(rows0, s_agg.at[id0], add=True)
                load_pair(2 * j + 2, is0, id0)
                pltpu.async_copy(x_hbm.at[is0], rows0, sem0)

                pltpu.make_async_copy(x_hbm.at[is1], rows1, sem1).wait()
                pltpu.sync_copy(rows1, s_agg.at[id1], add=True)

                @pl.when(j < _HALF - 1)
                def _():
                    load_pair(2 * j + 3, is1, id1)
                    pltpu.async_copy(x_hbm.at[is1], rows1, sem1)
                return 0
            lax.fori_loop(0, _HALF, p1, 0)

            pltpu.make_async_copy(x_hbm.at[is0], rows0, sem0).wait()
            pltpu.sync_copy(rows0, s_agg.at[id0], add=True)

            plsc.subcore_barrier()
            writeout(s_agg, agg_hbm)
            plsc.subcore_barrier()

            fill_rows0(jnp.zeros((16,), jnp.float32))
            zero_acc(s_agg)
            plsc.subcore_barrier()
            fill_rows0(jnp.full((16,), 1.0, jnp.float32))

            def p2(j, _):
                @pl.when(j > 0)
                def _():
                    pltpu.make_async_copy(rows0, s_agg.at[id0], sem0).wait()
                load_dst(2 * j, id0)
                pltpu.async_copy(rows0, s_agg.at[id0], sem0, add=True)

                @pl.when(j < _HALF)
                def _():
                    @pl.when(j > 0)
                    def _():
                        pltpu.make_async_copy(rows0, s_agg.at[id1], sem1).wait()
                    load_dst(2 * j + 1, id1)
                    pltpu.async_copy(rows0, s_agg.at[id1], sem1, add=True)
                return 0
            lax.fori_loop(0, _HALF + 1, p2, 0)

            pltpu.make_async_copy(rows0, s_agg.at[id0], sem0).wait()
            pltpu.make_async_copy(rows0, s_agg.at[id1], sem1).wait()

            plsc.subcore_barrier()
            writeout(s_agg, deg_hbm)

        @pl.when(cid == 0)
        def _():
            run(sab_hbm, dab_hbm, xa_hbm, aggB_hbm, degB_hbm)

        @pl.when(cid == 1)
        def _():
            run(sba_hbm, dba_hbm, xb_hbm, aggA_hbm, degA_hbm)

    return body(x_A, x_B, src_ab, dst_ab, src_ba, dst_ba)


def _tc_dense(x_A, x_B, agg_A, agg_B, deg_A, deg_B,
              Ws_A, Ws_B, Wc_ab, Wc_ba, bs_A, bs_B, bc_ab, bc_ba,
              wl_A, wr_A, wl_B, wr_B, consts):
    BLK = 1000
    grid = _N // BLK

    def elu(x):
        return jnp.where(x > 0, x, jnp.exp(jnp.minimum(x, 0.0)) - 1.0)

    def body(xa, xb, ga, gb, da, db, WsA, WsB, Wcab, Wcba,
             bsA, bsB, bcab, bcba, wlA, wrA, wlB, wrB, cst, out):
        clA, crA, clB, crB = cst[0, 0], cst[0, 1], cst[0, 2], cst[0, 3]
        dot = functools.partial(jnp.dot, precision=jax.lax.Precision.HIGHEST,
                                preferred_element_type=jnp.float32)
        za = dot(xa[...], WsA[...]) + bsA[...]
        zb = dot(xb[...], WsB[...]) + bsB[...]
        hlA = jnp.sum(za * wlA[...], axis=1, keepdims=True) + clA
        hrA = jnp.sum(za * wrA[...], axis=1, keepdims=True) + crA
        hlB = jnp.sum(zb * wlB[...], axis=1, keepdims=True) + clB
        hrB = jnp.sum(zb * wrB[...], axis=1, keepdims=True) + crB
        dA = dot(ga[...] / jnp.maximum(da[...], 1.0), Wcba[...]) + bcba[...]
        dB = dot(gb[...] / jnp.maximum(db[...], 1.0), Wcab[...]) + bcab[...]
        attsA = elu(hlA + hrA)
        attsB = elu(hlB + hrB)
        eA = elu(jnp.sum(dA * wlA[...], axis=1, keepdims=True) + clA + hrA)
        eB = elu(jnp.sum(dB * wlB[...], axis=1, keepdims=True) + clB + hrB)
        mA = jnp.maximum(attsA, eA)
        p0A, p1A = jnp.exp(attsA - mA), jnp.exp(eA - mA)
        sA = p0A + p1A
        mB = jnp.maximum(attsB, eB)
        p0B, p1B = jnp.exp(attsB - mB), jnp.exp(eB - mB)
        sB = p0B + p1B
        out[0] = elu(za * (p0A / sA) + dA * (p1A / sA))
        out[1] = elu(zb * (p0B / sB) + dB * (p1B / sB))

    row_blk = pl.BlockSpec((BLK, _F), lambda i: (i, 0))
    col_blk = pl.BlockSpec((BLK, 1), lambda i: (i, 0))
    w_blk = pl.BlockSpec((_F, _F), lambda i: (0, 0))
    v_blk = pl.BlockSpec((1, _F), lambda i: (0, 0))
    c_blk = pl.BlockSpec((1, 4), lambda i: (0, 0))

    return pl.pallas_call(
        body,
        grid=(grid,),
        in_specs=[row_blk, row_blk, row_blk, row_blk, col_blk, col_blk,
                  w_blk, w_blk, w_blk, w_blk,
                  v_blk, v_blk, v_blk, v_blk,
                  v_blk, v_blk, v_blk, v_blk, c_blk],
        out_specs=pl.BlockSpec((2, BLK, _F), lambda i: (0, i, 0)),
        out_shape=jax.ShapeDtypeStruct((2, _N, _F), jnp.float32),
    )(x_A, x_B, agg_A, agg_B, deg_A, deg_B,
      Ws_A, Ws_B, Wc_ab, Wc_ba, bs_A, bs_B, bc_ab, bc_ba,
      wl_A, wr_A, wl_B, wr_B, consts)


def kernel(x_A, x_B, edge_ab, edge_ba, Ws_A, bs_A, Ws_B, bs_B, Wq_A, bq_A,
           Wq_B, bq_B, Wk_A, bk_A, Wk_B, bk_B, Wal_A, bal_A, Wal_B, bal_B,
           War_A, bar_A, War_B, bar_B, Wc_ab, bc_ab, Wc_ba, bc_ba):
    wl_A = (Wk_A @ Wal_A).reshape(1, _F)
    wl_B = (Wk_B @ Wal_B).reshape(1, _F)
    wr_A = (Wq_A @ War_A).reshape(1, _F)
    wr_B = (Wq_B @ War_B).reshape(1, _F)
    cl_A = bk_A @ Wal_A + bal_A
    cl_B = bk_B @ Wal_B + bal_B
    cr_A = bq_A @ War_A + bar_A
    cr_B = bq_B @ War_B + bar_B
    consts = jnp.concatenate([cl_A, cr_A, cl_B, cr_B]).reshape(1, 4)

    agg_A, agg_B, deg_A, deg_B = _sc_aggregate(
        x_A, x_B, edge_ab[0], edge_ab[1], edge_ba[0], edge_ba[1])

    return _tc_dense(
        x_A, x_B, agg_A[:_N], agg_B[:_N],
        deg_A[:_N, :1], deg_B[:_N, :1],
        Ws_A, Ws_B, Wc_ab, Wc_ba,
        bs_A.reshape(1, _F), bs_B.reshape(1, _F),
        bc_ab.reshape(1, _F), bc_ba.reshape(1, _F),
        wl_A, wr_A, wl_B, wr_B, consts)


# speedup vs baseline: 9.1560x; 1.1590x over previous
import functools

import jax
import jax.numpy as jnp
from jax import lax
from jax.experimental import pallas as pl
from jax.experimental.pallas import tpu as pltpu
from jax.experimental.pallas import tpu_sc as plsc

_N = 10000
_E = 320000
_F = 128
_NSUB = 16
_EPW = _E // _NSUB
_C = 160
_NCH = _EPW // _C
_NPAD = 10240
_RPW = _NPAD // _NSUB


_HALF = (_NCH - 1) // 2


def _sc_aggregate(x_A, x_B, src_ab, dst_ab, src_ba, dst_ba):
    mesh = plsc.VectorSubcoreMesh(core_axis_name="c", subcore_axis_name="s")

    @functools.partial(
        pl.kernel,
        out_type=[
            jax.ShapeDtypeStruct((_NPAD, _F), jnp.float32),
            jax.ShapeDtypeStruct((_NPAD, _F), jnp.float32),
            jax.ShapeDtypeStruct((_NPAD, _F), jnp.float32),
            jax.ShapeDtypeStruct((_NPAD, _F), jnp.float32),
        ],
        mesh=mesh,
        scratch_types=[
            pltpu.VMEM((_C,), jnp.int32),
            pltpu.VMEM((_C,), jnp.int32),
            pltpu.VMEM((_C,), jnp.int32),
            pltpu.VMEM((_C,), jnp.int32),
            pltpu.VMEM((_C, _F), jnp.float32),
            pltpu.VMEM((_C, _F), jnp.float32),
            pltpu.VMEM_SHARED((_NPAD, _F), jnp.float32),
            pltpu.SemaphoreType.DMA,
            pltpu.SemaphoreType.DMA,
        ],
    )
    def body(xa_hbm, xb_hbm, sab_hbm, dab_hbm, sba_hbm, dba_hbm,
             aggA_hbm, aggB_hbm, degA_hbm, degB_hbm,
             is0, id0, is1, id1, rows0, rows1, s_agg, sem0, sem1):
        cid = lax.axis_index("c")
        sid = lax.axis_index("s")

        def fill_rows0(val16):
            def fill(r, _):
                for j in range(_F // 16):
                    rows0[r, pl.ds(j * 16, 16)] = val16
                return 0
            lax.fori_loop(0, _C, fill, 0)

        def zero_acc(s_agg):
            def z(i, _):
                pltpu.sync_copy(rows0,
                                s_agg.at[pl.ds(sid * _RPW + i * _C, _C), :])
                return 0
            lax.fori_loop(0, _RPW // _C, z, 0)

        def writeout(s_agg, dst_hbm):
            pltpu.sync_copy(s_agg.at[pl.ds(sid * _RPW, _RPW), :],
                            dst_hbm.at[pl.ds(sid * _RPW, _RPW), :])

        def run(src_hbm, dst_hbm, x_hbm, agg_hbm, deg_hbm):
            def load_pair(k, is_b, id_b):
                base = sid * _EPW + k * _C
                pltpu.sync_copy(src_hbm.at[pl.ds(base, _C)], is_b)
                pltpu.sync_copy(dst_hbm.at[pl.ds(base, _C)], id_b)

            def load_pair2(k, is_b):
                base = sid * _EPW + k * _C
                pltpu.sync_copy(src_hbm.at[pl.ds(base, _C)], is_b)

            def load_dst(k, id_b):
                base = sid * _EPW + k * _C
                pltpu.sync_copy(dst_hbm.at[pl.ds(base, _C)], id_b)

            fill_rows0(jnp.zeros((16,), jnp.float32))
            zero_acc(s_agg)
            plsc.subcore_barrier()

            load_pair(0, is0, id0)
            pltpu.async_copy(x_hbm.at[is0], rows0, sem0)
            load_pair(1, is1, id1)
            pltpu.async_copy(x_hbm.at[is1], rows1, sem1)

            def p1(j, _):
                pltpu.make_async_copy(x_hbm.at[is0], rows0, sem0).wait()
                pltpu.async_copy(rows0, s_agg.at[id0], sem0, add=True)
                load_pair2(2 * j + 2, is0)
                pltpu.make_async_copy(rows0, s_agg.at[id0], sem0).wait()
                pltpu.sync_copy(dst_hbm.at[pl.ds(sid * _EPW + (2 * j + 2) * _C, _C)], id0)
                pltpu.async_copy(x_hbm.at[is0], rows0, sem0)

                pltpu.make_async_copy(x_hbm.at[is1], rows1, sem1).wait()
                pltpu.async_copy(rows1, s_agg.at[id1], sem1, add=True)

                @pl.when(j < _HALF - 1)
                def _():
                    load_pair2(2 * j + 3, is1)
                pltpu.make_async_copy(rows1, s_agg.at[id1], sem1).wait()

                @pl.when(j < _HALF - 1)
                def _():
                    pltpu.sync_copy(dst_hbm.at[pl.ds(sid * _EPW + (2 * j + 3) * _C, _C)], id1)
                    pltpu.async_copy(x_hbm.at[is1], rows1, sem1)
                return 0
            lax.fori_loop(0, _HALF, p1, 0)

            pltpu.make_async_copy(x_hbm.at[is0], rows0, sem0).wait()
            pltpu.sync_copy(rows0, s_agg.at[id0], add=True)

            plsc.subcore_barrier()
            writeout(s_agg, agg_hbm)
            plsc.subcore_barrier()

            fill_rows0(jnp.zeros((16,), jnp.float32))
            zero_acc(s_agg)
            plsc.subcore_barrier()
            fill_rows0(jnp.full((16,), 1.0, jnp.float32))

            def p2(j, _):
                @pl.when(j > 0)
                def _():
                    pltpu.make_async_copy(rows0, s_agg.at[id0], sem0).wait()
                load_dst(2 * j, id0)
                pltpu.async_copy(rows0, s_agg.at[id0], sem0, add=True)

                @pl.when(j < _HALF)
                def _():
                    @pl.when(j > 0)
                    def _():
                        pltpu.make_async_copy(rows0, s_agg.at[id1], sem1).wait()
                    load_dst(2 * j + 1, id1)
                    pltpu.async_copy(rows0, s_agg.at[id1], sem1, add=True)
                return 0
            lax.fori_loop(0, _HALF + 1, p2, 0)

            pltpu.make_async_copy(rows0, s_agg.at[id0], sem0).wait()
            pltpu.make_async_copy(rows0, s_agg.at[id1], sem1).wait()

            plsc.subcore_barrier()
            writeout(s_agg, deg_hbm)

        @pl.when(cid == 0)
        def _():
            run(sab_hbm, dab_hbm, xa_hbm, aggB_hbm, degB_hbm)

        @pl.when(cid == 1)
        def _():
            run(sba_hbm, dba_hbm, xb_hbm, aggA_hbm, degA_hbm)

    return body(x_A, x_B, src_ab, dst_ab, src_ba, dst_ba)


def _tc_dense(x_A, x_B, agg_A, agg_B, deg_A, deg_B,
              Ws_A, Ws_B, Wc_ab, Wc_ba, bs_A, bs_B, bc_ab, bc_ba,
              wl_A, wr_A, wl_B, wr_B, consts):
    BLK = 1000
    grid = _N // BLK

    def elu(x):
        return jnp.where(x > 0, x, jnp.exp(jnp.minimum(x, 0.0)) - 1.0)

    def body(xa, xb, ga, gb, da, db, WsA, WsB, Wcab, Wcba,
             bsA, bsB, bcab, bcba, wlA, wrA, wlB, wrB, cst, out):
        clA, crA, clB, crB = cst[0, 0], cst[0, 1], cst[0, 2], cst[0, 3]
        dot = functools.partial(jnp.dot, preferred_element_type=jnp.float32)
        za = dot(xa[...], WsA[...]) + bsA[...]
        zb = dot(xb[...], WsB[...]) + bsB[...]
        hlA = jnp.sum(za * wlA[...], axis=1, keepdims=True) + clA
        hrA = jnp.sum(za * wrA[...], axis=1, keepdims=True) + crA
        hlB = jnp.sum(zb * wlB[...], axis=1, keepdims=True) + clB
        hrB = jnp.sum(zb * wrB[...], axis=1, keepdims=True) + crB
        dA = dot(ga[...] / jnp.maximum(da[..., 0:1], 1.0), Wcba[...]) + bcba[...]
        dB = dot(gb[...] / jnp.maximum(db[..., 0:1], 1.0), Wcab[...]) + bcab[...]
        attsA = elu(hlA + hrA)
        attsB = elu(hlB + hrB)
        eA = elu(jnp.sum(dA * wlA[...], axis=1, keepdims=True) + clA + hrA)
        eB = elu(jnp.sum(dB * wlB[...], axis=1, keepdims=True) + clB + hrB)
        mA = jnp.maximum(attsA, eA)
        p0A, p1A = jnp.exp(attsA - mA), jnp.exp(eA - mA)
        sA = p0A + p1A
        mB = jnp.maximum(attsB, eB)
        p0B, p1B = jnp.exp(attsB - mB), jnp.exp(eB - mB)
        sB = p0B + p1B
        out[0] = elu(za * (p0A / sA) + dA * (p1A / sA))
        out[1] = elu(zb * (p0B / sB) + dB * (p1B / sB))

    row_blk = pl.BlockSpec((BLK, _F), lambda i: (i, 0))
    col_blk = pl.BlockSpec((BLK, _F), lambda i: (i, 0))
    w_blk = pl.BlockSpec((_F, _F), lambda i: (0, 0))
    v_blk = pl.BlockSpec((1, _F), lambda i: (0, 0))
    c_blk = pl.BlockSpec((1, 4), lambda i: (0, 0))

    return pl.pallas_call(
        body,
        grid=(grid,),
        in_specs=[row_blk, row_blk, row_blk, row_blk, col_blk, col_blk,
                  w_blk, w_blk, w_blk, w_blk,
                  v_blk, v_blk, v_blk, v_blk,
                  v_blk, v_blk, v_blk, v_blk, c_blk],
        out_specs=pl.BlockSpec((2, BLK, _F), lambda i: (0, i, 0)),
        out_shape=jax.ShapeDtypeStruct((2, _N, _F), jnp.float32),
    )(x_A, x_B, agg_A, agg_B, deg_A, deg_B,
      Ws_A, Ws_B, Wc_ab, Wc_ba, bs_A, bs_B, bc_ab, bc_ba,
      wl_A, wr_A, wl_B, wr_B, consts)


def kernel(x_A, x_B, edge_ab, edge_ba, Ws_A, bs_A, Ws_B, bs_B, Wq_A, bq_A,
           Wq_B, bq_B, Wk_A, bk_A, Wk_B, bk_B, Wal_A, bal_A, Wal_B, bal_B,
           War_A, bar_A, War_B, bar_B, Wc_ab, bc_ab, Wc_ba, bc_ba):
    wl_A = (Wk_A @ Wal_A).reshape(1, _F)
    wl_B = (Wk_B @ Wal_B).reshape(1, _F)
    wr_A = (Wq_A @ War_A).reshape(1, _F)
    wr_B = (Wq_B @ War_B).reshape(1, _F)
    cl_A = bk_A @ Wal_A + bal_A
    cl_B = bk_B @ Wal_B + bal_B
    cr_A = bq_A @ War_A + bar_A
    cr_B = bq_B @ War_B + bar_B
    consts = jnp.concatenate([cl_A, cr_A, cl_B, cr_B]).reshape(1, 4)

    agg_A, agg_B, deg_A, deg_B = _sc_aggregate(
        x_A, x_B, edge_ab[0], edge_ab[1], edge_ba[0], edge_ba[1])

    return _tc_dense(
        x_A, x_B, agg_A[:_N], agg_B[:_N],
        deg_A[:_N, :1], deg_B[:_N, :1],
        Ws_A, Ws_B, Wc_ab, Wc_ba,
        bs_A.reshape(1, _F), bs_B.reshape(1, _F),
        bc_ab.reshape(1, _F), bc_ba.reshape(1, _F),
        wl_A, wr_A, wl_B, wr_B, consts)


# speedup vs baseline: 9.1705x; 1.0016x over previous
import functools

import jax
import jax.numpy as jnp
from jax import lax
from jax.experimental import pallas as pl
from jax.experimental.pallas import tpu as pltpu
from jax.experimental.pallas import tpu_sc as plsc

_N = 10000
_E = 320000
_F = 128
_NSUB = 16
_EPW = _E // _NSUB
_C = 160
_NCH = _EPW // _C
_NPAD = 10240
_RPW = _NPAD // _NSUB


_HALF = (_NCH - 1) // 2


def _sc_aggregate(x_A, x_B, src_ab, dst_ab, src_ba, dst_ba):
    mesh = plsc.VectorSubcoreMesh(core_axis_name="c", subcore_axis_name="s")

    @functools.partial(
        pl.kernel,
        out_type=[
            jax.ShapeDtypeStruct((_NPAD, _F), jnp.float32),
            jax.ShapeDtypeStruct((_NPAD, _F), jnp.float32),
            jax.ShapeDtypeStruct((_NPAD, _F), jnp.float32),
            jax.ShapeDtypeStruct((_NPAD, _F), jnp.float32),
        ],
        mesh=mesh,
        scratch_types=[
            pltpu.VMEM((_C,), jnp.int32),
            pltpu.VMEM((_C,), jnp.int32),
            pltpu.VMEM((_C,), jnp.int32),
            pltpu.VMEM((_C,), jnp.int32),
            pltpu.VMEM((_C, _F), jnp.float32),
            pltpu.VMEM((_C, _F), jnp.float32),
            pltpu.VMEM_SHARED((_NPAD, _F), jnp.float32),
            pltpu.SemaphoreType.DMA,
            pltpu.SemaphoreType.DMA,
        ],
    )
    def body(xa_hbm, xb_hbm, sab_hbm, dab_hbm, sba_hbm, dba_hbm,
             aggA_hbm, aggB_hbm, degA_hbm, degB_hbm,
             is0, id0, is1, id1, rows0, rows1, s_agg, sem0, sem1):
        cid = lax.axis_index("c")
        sid = lax.axis_index("s")

        def fill_rows0(val16):
            def fill(r, _):
                for j in range(_F // 16):
                    rows0[r, pl.ds(j * 16, 16)] = val16
                return 0
            lax.fori_loop(0, _C, fill, 0)

        def zero_acc(s_agg):
            def z(i, _):
                pltpu.sync_copy(rows0,
                                s_agg.at[pl.ds(sid * _RPW + i * _C, _C), :])
                return 0
            lax.fori_loop(0, _RPW // _C, z, 0)

        def writeout(s_agg, dst_hbm):
            pltpu.sync_copy(s_agg.at[pl.ds(sid * _RPW, _RPW), :],
                            dst_hbm.at[pl.ds(sid * _RPW, _RPW), :])

        def run(src_hbm, dst_hbm, x_hbm, agg_hbm, deg_hbm):
            def load_pair(k, is_b, id_b):
                base = sid * _EPW + k * _C
                pltpu.sync_copy(src_hbm.at[pl.ds(base, _C)], is_b)
                pltpu.sync_copy(dst_hbm.at[pl.ds(base, _C)], id_b)

            def load_pair2(k, is_b):
                base = sid * _EPW + k * _C
                pltpu.sync_copy(src_hbm.at[pl.ds(base, _C)], is_b)

            def load_dst(k, id_b):
                base = sid * _EPW + k * _C
                pltpu.sync_copy(dst_hbm.at[pl.ds(base, _C)], id_b)

            fill_rows0(jnp.zeros((16,), jnp.float32))
            zero_acc(s_agg)
            plsc.subcore_barrier()

            load_pair(0, is0, id0)
            pltpu.async_copy(x_hbm.at[is0], rows0, sem0)
            load_pair(1, is1, id1)
            pltpu.async_copy(x_hbm.at[is1], rows1, sem1)

            def p1(j, _):
                pltpu.make_async_copy(x_hbm.at[is0], rows0, sem0).wait()
                pltpu.async_copy(rows0, s_agg.at[id0], sem0, add=True)
                load_pair2(2 * j + 2, is0)
                pltpu.make_async_copy(rows0, s_agg.at[id0], sem0).wait()
                pltpu.sync_copy(dst_hbm.at[pl.ds(sid * _EPW + (2 * j + 2) * _C, _C)], id0)
                pltpu.async_copy(x_hbm.at[is0], rows0, sem0)

                pltpu.make_async_copy(x_hbm.at[is1], rows1, sem1).wait()
                pltpu.async_copy(rows1, s_agg.at[id1], sem1, add=True)

                @pl.when(j < _HALF - 1)
                def _():
                    load_pair2(2 * j + 3, is1)
                pltpu.make_async_copy(rows1, s_agg.at[id1], sem1).wait()

                @pl.when(j < _HALF - 1)
                def _():
                    pltpu.sync_copy(dst_hbm.at[pl.ds(sid * _EPW + (2 * j + 3) * _C, _C)], id1)
                    pltpu.async_copy(x_hbm.at[is1], rows1, sem1)
                return 0
            lax.fori_loop(0, _HALF, p1, 0)

            pltpu.make_async_copy(x_hbm.at[is0], rows0, sem0).wait()
            pltpu.sync_copy(rows0, s_agg.at[id0], add=True)

            plsc.subcore_barrier()
            writeout(s_agg, agg_hbm)
            plsc.subcore_barrier()

            fill_rows0(jnp.zeros((16,), jnp.float32))
            zero_acc(s_agg)
            plsc.subcore_barrier()
            fill_rows0(jnp.full((16,), 1.0, jnp.float32))

            def p2(j, _):
                @pl.when(j > 0)
                def _():
                    pltpu.make_async_copy(rows0, s_agg.at[id0], sem0).wait()
                load_dst(2 * j, id0)
                pltpu.async_copy(rows0, s_agg.at[id0], sem0, add=True)

                @pl.when(j < _HALF)
                def _():
                    @pl.when(j > 0)
                    def _():
                        pltpu.make_async_copy(rows0, s_agg.at[id1], sem1).wait()
                    load_dst(2 * j + 1, id1)
                    pltpu.async_copy(rows0, s_agg.at[id1], sem1, add=True)
                return 0
            lax.fori_loop(0, _HALF + 1, p2, 0)

            pltpu.make_async_copy(rows0, s_agg.at[id0], sem0).wait()
            pltpu.make_async_copy(rows0, s_agg.at[id1], sem1).wait()

            plsc.subcore_barrier()
            writeout(s_agg, deg_hbm)

        @pl.when(cid == 0)
        def _():
            run(sab_hbm, dab_hbm, xa_hbm, aggB_hbm, degB_hbm)

        @pl.when(cid == 1)
        def _():
            run(sba_hbm, dba_hbm, xb_hbm, aggA_hbm, degA_hbm)

    return body(x_A, x_B, src_ab, dst_ab, src_ba, dst_ba)


def _tc_dense(x_A, x_B, agg_A, agg_B, deg_A, deg_B,
              Ws_A, Ws_B, Wc_ab, Wc_ba, bs_A, bs_B, bc_ab, bc_ba,
              wl_A, wr_A, wl_B, wr_B, consts):
    BLK = 2000
    grid = _N // BLK

    def elu(x):
        return jnp.where(x > 0, x, jnp.exp(jnp.minimum(x, 0.0)) - 1.0)

    def body(xa, xb, ga, gb, da, db, WsA, WsB, Wcab, Wcba,
             bsA, bsB, bcab, bcba, wlA, wrA, wlB, wrB, cst, out):
        clA, crA, clB, crB = cst[0, 0], cst[0, 1], cst[0, 2], cst[0, 3]
        dot = functools.partial(jnp.dot, preferred_element_type=jnp.float32)
        za = dot(xa[...], WsA[...]) + bsA[...]
        zb = dot(xb[...], WsB[...]) + bsB[...]
        hlA = jnp.sum(za * wlA[...], axis=1, keepdims=True) + clA
        hrA = jnp.sum(za * wrA[...], axis=1, keepdims=True) + crA
        hlB = jnp.sum(zb * wlB[...], axis=1, keepdims=True) + clB
        hrB = jnp.sum(zb * wrB[...], axis=1, keepdims=True) + crB
        dA = dot(ga[...] / jnp.maximum(da[..., 0:1], 1.0), Wcba[...]) + bcba[...]
        dB = dot(gb[...] / jnp.maximum(db[..., 0:1], 1.0), Wcab[...]) + bcab[...]
        attsA = elu(hlA + hrA)
        attsB = elu(hlB + hrB)
        eA = elu(jnp.sum(dA * wlA[...], axis=1, keepdims=True) + clA + hrA)
        eB = elu(jnp.sum(dB * wlB[...], axis=1, keepdims=True) + clB + hrB)
        mA = jnp.maximum(attsA, eA)
        p0A, p1A = jnp.exp(attsA - mA), jnp.exp(eA - mA)
        sA = p0A + p1A
        mB = jnp.maximum(attsB, eB)
        p0B, p1B = jnp.exp(attsB - mB), jnp.exp(eB - mB)
        sB = p0B + p1B
        out[0] = elu(za * (p0A / sA) + dA * (p1A / sA))
        out[1] = elu(zb * (p0B / sB) + dB * (p1B / sB))

    row_blk = pl.BlockSpec((BLK, _F), lambda i: (i, 0))
    col_blk = pl.BlockSpec((BLK, _F), lambda i: (i, 0))
    w_blk = pl.BlockSpec((_F, _F), lambda i: (0, 0))
    v_blk = pl.BlockSpec((1, _F), lambda i: (0, 0))
    c_blk = pl.BlockSpec((1, 4), lambda i: (0, 0))

    return pl.pallas_call(
        body,
        grid=(grid,),
        in_specs=[row_blk, row_blk, row_blk, row_blk, col_blk, col_blk,
                  w_blk, w_blk, w_blk, w_blk,
                  v_blk, v_blk, v_blk, v_blk,
                  v_blk, v_blk, v_blk, v_blk, c_blk],
        out_specs=pl.BlockSpec((2, BLK, _F), lambda i: (0, i, 0)),
        out_shape=jax.ShapeDtypeStruct((2, _N, _F), jnp.float32),
    )(x_A, x_B, agg_A, agg_B, deg_A, deg_B,
      Ws_A, Ws_B, Wc_ab, Wc_ba, bs_A, bs_B, bc_ab, bc_ba,
      wl_A, wr_A, wl_B, wr_B, consts)


def kernel(x_A, x_B, edge_ab, edge_ba, Ws_A, bs_A, Ws_B, bs_B, Wq_A, bq_A,
           Wq_B, bq_B, Wk_A, bk_A, Wk_B, bk_B, Wal_A, bal_A, Wal_B, bal_B,
           War_A, bar_A, War_B, bar_B, Wc_ab, bc_ab, Wc_ba, bc_ba):
    wl_A = (Wk_A @ Wal_A).reshape(1, _F)
    wl_B = (Wk_B @ Wal_B).reshape(1, _F)
    wr_A = (Wq_A @ War_A).reshape(1, _F)
    wr_B = (Wq_B @ War_B).reshape(1, _F)
    cl_A = bk_A @ Wal_A + bal_A
    cl_B = bk_B @ Wal_B + bal_B
    cr_A = bq_A @ War_A + bar_A
    cr_B = bq_B @ War_B + bar_B
    consts = jnp.concatenate([cl_A, cr_A, cl_B, cr_B]).reshape(1, 4)

    agg_A, agg_B, deg_A, deg_B = _sc_aggregate(
        x_A, x_B, edge_ab[0], edge_ab[1], edge_ba[0], edge_ba[1])

    return _tc_dense(
        x_A, x_B, agg_A[:_N], agg_B[:_N],
        deg_A[:_N, :1], deg_B[:_N, :1],
        Ws_A, Ws_B, Wc_ab, Wc_ba,
        bs_A.reshape(1, _F), bs_B.reshape(1, _F),
        bc_ab.reshape(1, _F), bc_ba.reshape(1, _F),
        wl_A, wr_A, wl_B, wr_B, consts)


# speedup vs baseline: 9.3081x; 1.0150x over previous
import functools

import jax
import jax.numpy as jnp
from jax import lax
from jax.experimental import pallas as pl
from jax.experimental.pallas import tpu as pltpu
from jax.experimental.pallas import tpu_sc as plsc

_N = 10000
_E = 320000
_F = 128
_NSUB = 16
_EPW = _E // _NSUB
_C = 160
_NCH = _EPW // _C
_NPAD = 10240
_RPW = _NPAD // _NSUB


_HALF = (_NCH - 1) // 2


def _sc_aggregate(x_A, x_B, src_ab, dst_ab, src_ba, dst_ba):
    mesh = plsc.VectorSubcoreMesh(core_axis_name="c", subcore_axis_name="s")

    @functools.partial(
        pl.kernel,
        out_type=[
            jax.ShapeDtypeStruct((_NPAD, _F), jnp.float32),
            jax.ShapeDtypeStruct((_NPAD, _F), jnp.float32),
            jax.ShapeDtypeStruct((_NPAD, _F), jnp.float32),
            jax.ShapeDtypeStruct((_NPAD, _F), jnp.float32),
        ],
        mesh=mesh,
        scratch_types=[
            pltpu.VMEM((_C,), jnp.int32),
            pltpu.VMEM((_C,), jnp.int32),
            pltpu.VMEM((_C,), jnp.int32),
            pltpu.VMEM((_C,), jnp.int32),
            pltpu.VMEM((_C, _F), jnp.float32),
            pltpu.VMEM((_C, _F), jnp.float32),
            pltpu.VMEM_SHARED((_NPAD, _F), jnp.float32),
            pltpu.SemaphoreType.DMA,
            pltpu.SemaphoreType.DMA,
        ],
    )
    def body(xa_hbm, xb_hbm, sab_hbm, dab_hbm, sba_hbm, dba_hbm,
             aggA_hbm, aggB_hbm, degA_hbm, degB_hbm,
             is0, id0, is1, id1, rows0, rows1, s_agg, sem0, sem1):
        cid = lax.axis_index("c")
        sid = lax.axis_index("s")

        def fill_rows0(val16):
            def fill(r, _):
                for j in range(_F // 16):
                    rows0[r, pl.ds(j * 16, 16)] = val16
                return 0
            lax.fori_loop(0, _C, fill, 0)

        def zero_acc(s_agg):
            def z(i, _):
                pltpu.sync_copy(rows0,
                                s_agg.at[pl.ds(sid * _RPW + i * _C, _C), :])
                return 0
            lax.fori_loop(0, _RPW // _C, z, 0)

        def writeout(s_agg, dst_hbm):
            pltpu.sync_copy(s_agg.at[pl.ds(sid * _RPW, _RPW), :],
                            dst_hbm.at[pl.ds(sid * _RPW, _RPW), :])

        def run(src_hbm, dst_hbm, x_hbm, agg_hbm, deg_hbm):
            def load_pair(k, is_b, id_b):
                base = sid * _EPW + k * _C
                pltpu.sync_copy(src_hbm.at[pl.ds(base, _C)], is_b)
                pltpu.sync_copy(dst_hbm.at[pl.ds(base, _C)], id_b)

            def load_pair2(k, is_b):
                base = sid * _EPW + k * _C
                pltpu.sync_copy(src_hbm.at[pl.ds(base, _C)], is_b)

            def load_dst(k, id_b):
                base = sid * _EPW + k * _C
                pltpu.sync_copy(dst_hbm.at[pl.ds(base, _C)], id_b)

            fill_rows0(jnp.zeros((16,), jnp.float32))
            zero_acc(s_agg)
            plsc.subcore_barrier()

            load_pair(0, is0, id0)
            pltpu.async_copy(x_hbm.at[is0], rows0, sem0)
            load_pair(1, is1, id1)
            pltpu.async_copy(x_hbm.at[is1], rows1, sem1)

            def p1(j, _):
                pltpu.make_async_copy(x_hbm.at[is0], rows0, sem0).wait()
                pltpu.async_copy(rows0, s_agg.at[id0], sem0, add=True)
                load_pair2(2 * j + 2, is0)
                pltpu.make_async_copy(rows0, s_agg.at[id0], sem0).wait()
                pltpu.sync_copy(dst_hbm.at[pl.ds(sid * _EPW + (2 * j + 2) * _C, _C)], id0)
                pltpu.async_copy(x_hbm.at[is0], rows0, sem0)

                pltpu.make_async_copy(x_hbm.at[is1], rows1, sem1).wait()
                pltpu.async_copy(rows1, s_agg.at[id1], sem1, add=True)

                @pl.when(j < _HALF - 1)
                def _():
                    load_pair2(2 * j + 3, is1)
                pltpu.make_async_copy(rows1, s_agg.at[id1], sem1).wait()

                @pl.when(j < _HALF - 1)
                def _():
                    pltpu.sync_copy(dst_hbm.at[pl.ds(sid * _EPW + (2 * j + 3) * _C, _C)], id1)
                    pltpu.async_copy(x_hbm.at[is1], rows1, sem1)
                return 0
            lax.fori_loop(0, _HALF, p1, 0)

            pltpu.make_async_copy(x_hbm.at[is0], rows0, sem0).wait()
            pltpu.sync_copy(rows0, s_agg.at[id0], add=True)

            plsc.subcore_barrier()
            writeout(s_agg, agg_hbm)
            plsc.subcore_barrier()

            fill_rows0(jnp.zeros((16,), jnp.float32))
            zero_acc(s_agg)
            plsc.subcore_barrier()
            fill_rows0(jnp.full((16,), 1.0, jnp.float32))

            def p2(j, _):
                @pl.when(j > 0)
                def _():
                    pltpu.make_async_copy(rows0, s_agg.at[id0], sem0).wait()
                load_dst(2 * j, id0)
                pltpu.async_copy(rows0, s_agg.at[id0], sem0, add=True)

                @pl.when(j < _HALF)
                def _():
                    @pl.when(j > 0)
                    def _():
                        pltpu.make_async_copy(rows0, s_agg.at[id1], sem1).wait()
                    load_dst(2 * j + 1, id1)
                    pltpu.async_copy(rows0, s_agg.at[id1], sem1, add=True)
                return 0
            lax.fori_loop(0, _HALF + 1, p2, 0)

            pltpu.make_async_copy(rows0, s_agg.at[id0], sem0).wait()
            pltpu.make_async_copy(rows0, s_agg.at[id1], sem1).wait()

            plsc.subcore_barrier()
            writeout(s_agg, deg_hbm)

        @pl.when(cid == 0)
        def _():
            run(sab_hbm, dab_hbm, xa_hbm, aggB_hbm, degB_hbm)

        @pl.when(cid == 1)
        def _():
            run(sba_hbm, dba_hbm, xb_hbm, aggA_hbm, degA_hbm)

    return body(x_A, x_B, src_ab, dst_ab, src_ba, dst_ba)


def _tc_pre(x_A, x_B, Ws_A, Ws_B, bs_A, bs_B, wl_A, wr_A, wl_B, wr_B, consts):
    BLK = 2000
    grid = _N // BLK

    def elu(x):
        return jnp.where(x > 0, x, jnp.exp(jnp.minimum(x, 0.0)) - 1.0)

    def body(xa, xb, WsA, WsB, bsA, bsB, wlA, wrA, wlB, wrB, cst,
             za_o, zb_o, atA_o, atB_o, hrA_o, hrB_o):
        clA, crA, clB, crB = cst[0, 0], cst[0, 1], cst[0, 2], cst[0, 3]
        dot = functools.partial(jnp.dot, preferred_element_type=jnp.float32)
        za = dot(xa[...], WsA[...]) + bsA[...]
        zb = dot(xb[...], WsB[...]) + bsB[...]
        hlA = jnp.sum(za * wlA[...], axis=1, keepdims=True) + clA
        hrA = jnp.sum(za * wrA[...], axis=1, keepdims=True) + crA
        hlB = jnp.sum(zb * wlB[...], axis=1, keepdims=True) + clB
        hrB = jnp.sum(zb * wrB[...], axis=1, keepdims=True) + crB
        za_o[...] = za
        zb_o[...] = zb
        atA_o[...] = elu(hlA + hrA)
        atB_o[...] = elu(hlB + hrB)
        hrA_o[...] = hrA
        hrB_o[...] = hrB

    row_blk = pl.BlockSpec((BLK, _F), lambda i: (i, 0))
    w_blk = pl.BlockSpec((_F, _F), lambda i: (0, 0))
    v_blk = pl.BlockSpec((1, _F), lambda i: (0, 0))
    c_blk = pl.BlockSpec((1, 4), lambda i: (0, 0))
    n_blk = pl.BlockSpec((BLK, 1), lambda i: (i, 0))

    return pl.pallas_call(
        body,
        grid=(grid,),
        in_specs=[row_blk, row_blk, w_blk, w_blk, v_blk, v_blk,
                  v_blk, v_blk, v_blk, v_blk, c_blk],
        out_specs=[row_blk, row_blk, n_blk, n_blk, n_blk, n_blk],
        out_shape=[jax.ShapeDtypeStruct((_N, _F), jnp.float32),
                   jax.ShapeDtypeStruct((_N, _F), jnp.float32),
                   jax.ShapeDtypeStruct((_N, 1), jnp.float32),
                   jax.ShapeDtypeStruct((_N, 1), jnp.float32),
                   jax.ShapeDtypeStruct((_N, 1), jnp.float32),
                   jax.ShapeDtypeStruct((_N, 1), jnp.float32)],
    )(x_A, x_B, Ws_A, Ws_B, bs_A, bs_B, wl_A, wr_A, wl_B, wr_B, consts)


def _tc_post(za, zb, attsA, attsB, hrA, hrB, agg_A, agg_B, deg_A, deg_B,
             Wc_ab, Wc_ba, bc_ab, bc_ba, wl_A, wl_B, consts):
    BLK = 2000
    grid = _N // BLK

    def elu(x):
        return jnp.where(x > 0, x, jnp.exp(jnp.minimum(x, 0.0)) - 1.0)

    def body(za_r, zb_r, atA_r, atB_r, hrA_r, hrB_r, ga, gb, da, db,
             Wcab, Wcba, bcab, bcba, wlA, wlB, cst, out):
        clA, clB = cst[0, 0], cst[0, 2]
        dot = functools.partial(jnp.dot, preferred_element_type=jnp.float32)
        dA = dot(ga[...] / jnp.maximum(da[..., 0:1], 1.0), Wcba[...]) + bcba[...]
        dB = dot(gb[...] / jnp.maximum(db[..., 0:1], 1.0), Wcab[...]) + bcab[...]
        eA = elu(jnp.sum(dA * wlA[...], axis=1, keepdims=True) + clA + hrA_r[...])
        eB = elu(jnp.sum(dB * wlB[...], axis=1, keepdims=True) + clB + hrB_r[...])
        attsA = atA_r[...]
        attsB = atB_r[...]
        mA = jnp.maximum(attsA, eA)
        p0A, p1A = jnp.exp(attsA - mA), jnp.exp(eA - mA)
        sA = p0A + p1A
        mB = jnp.maximum(attsB, eB)
        p0B, p1B = jnp.exp(attsB - mB), jnp.exp(eB - mB)
        sB = p0B + p1B
        out[0] = elu(za_r[...] * (p0A / sA) + dA * (p1A / sA))
        out[1] = elu(zb_r[...] * (p0B / sB) + dB * (p1B / sB))

    row_blk = pl.BlockSpec((BLK, _F), lambda i: (i, 0))
    n_blk = pl.BlockSpec((BLK, 1), lambda i: (i, 0))
    w_blk = pl.BlockSpec((_F, _F), lambda i: (0, 0))
    v_blk = pl.BlockSpec((1, _F), lambda i: (0, 0))
    c_blk = pl.BlockSpec((1, 4), lambda i: (0, 0))

    return pl.pallas_call(
        body,
        grid=(grid,),
        in_specs=[row_blk, row_blk, n_blk, n_blk, n_blk, n_blk,
                  row_blk, row_blk, row_blk, row_blk,
                  w_blk, w_blk, v_blk, v_blk, v_blk, v_blk, c_blk],
        out_specs=pl.BlockSpec((2, BLK, _F), lambda i: (0, i, 0)),
        out_shape=jax.ShapeDtypeStruct((2, _N, _F), jnp.float32),
    )(za, zb, attsA, attsB, hrA, hrB, agg_A, agg_B, deg_A, deg_B,
      Wc_ab, Wc_ba, bc_ab, bc_ba, wl_A, wl_B, consts)


def kernel(x_A, x_B, edge_ab, edge_ba, Ws_A, bs_A, Ws_B, bs_B, Wq_A, bq_A,
           Wq_B, bq_B, Wk_A, bk_A, Wk_B, bk_B, Wal_A, bal_A, Wal_B, bal_B,
           War_A, bar_A, War_B, bar_B, Wc_ab, bc_ab, Wc_ba, bc_ba):
    wl_A = (Wk_A @ Wal_A).reshape(1, _F)
    wl_B = (Wk_B @ Wal_B).reshape(1, _F)
    wr_A = (Wq_A @ War_A).reshape(1, _F)
    wr_B = (Wq_B @ War_B).reshape(1, _F)
    cl_A = bk_A @ Wal_A + bal_A
    cl_B = bk_B @ Wal_B + bal_B
    cr_A = bq_A @ War_A + bar_A
    cr_B = bq_B @ War_B + bar_B
    consts = jnp.concatenate([cl_A, cr_A, cl_B, cr_B]).reshape(1, 4)

    agg_A, agg_B, deg_A, deg_B = _sc_aggregate(
        x_A, x_B, edge_ab[0], edge_ab[1], edge_ba[0], edge_ba[1])

    za, zb, attsA, attsB, hrA, hrB = _tc_pre(
        x_A, x_B, Ws_A, Ws_B, bs_A.reshape(1, _F), bs_B.reshape(1, _F),
        wl_A, wr_A, wl_B, wr_B, consts)

    return _tc_post(
        za, zb, attsA, attsB, hrA, hrB, agg_A, agg_B, deg_A, deg_B,
        Wc_ab, Wc_ba, bc_ab.reshape(1, _F), bc_ba.reshape(1, _F),
        wl_A, wl_B, consts)
